# baseline (speedup 1.0000x reference)
import functools

import jax
import jax.numpy as jnp
from jax import lax
from jax.experimental import pallas as pl
from jax.experimental.pallas import tpu as pltpu
from jax.experimental.pallas import tpu_sc as plsc

B, S, E = 1024, 200, 64
V = 1000000
NW = 32
TOK_PER_W = (B * S) // NW
CHUNK = 200
NCHUNK = TOK_PER_W // CHUNK
NB = 4


def kernel(x, vocab_table, pos_table):
    mesh = plsc.VectorSubcoreMesh(core_axis_name="c", subcore_axis_name="s")

    @functools.partial(
        pl.kernel,
        out_type=jax.ShapeDtypeStruct((B * S, 128), jnp.float32),
        scratch_types=[
            pltpu.VMEM((TOK_PER_W,), jnp.int32),
            pltpu.VMEM((S * E,), jnp.float32),
            pltpu.VMEM((NB, CHUNK, 128), jnp.float32),
            pltpu.SemaphoreType.DMA((NB,)),
            pltpu.SemaphoreType.DMA((NB,)),
        ],
        mesh=mesh,
        compiler_params=pltpu.CompilerParams(use_tc_tiling_on_sc=False),
    )
    def k(x_hbm, vocab_hbm, pos_hbm, out_hbm, idx_v, pos_v, buf, gsems,
          ssems):
        wid = lax.axis_index("s") * 2 + lax.axis_index("c")
        tok0 = wid * TOK_PER_W
        pltpu.sync_copy(x_hbm.at[pl.ds(tok0, TOK_PER_W)], idx_v)
        pltpu.sync_copy(pos_hbm.at[pl.ds(0, S * E)], pos_v)

        def gather(ch, b):
            return pltpu.make_async_copy(
                vocab_hbm.at[idx_v.at[pl.ds(ch * CHUNK, CHUNK)]],
                buf.at[b], gsems.at[b],
            )

        def store(b, ch):
            return pltpu.make_async_copy(
                buf.at[b], out_hbm.at[pl.ds(tok0 + ch * CHUNK, CHUNK)],
                ssems.at[b],
            )

        for b in range(NB):
            gather(b, b).start()

        for ch in range(NCHUNK):
            b = ch % NB
            if 0 < ch and ch - 1 + NB < NCHUNK:
                pb = (ch - 1) % NB
                store(pb, ch - 1).wait()
                gather(ch - 1 + NB, pb).start()
            gather(ch, b).wait()

            @plsc.parallel_loop(0, CHUNK, 1, unroll=4)
            def add_body(i, b=b):
                for v in range(E // 16):
                    buf[b, i, pl.ds(v * 16, 16)] = (
                        buf[b, i, pl.ds(v * 16, 16)]
                        + pos_v[pl.ds(i * E + v * 16, 16)]
                    )
            store(b, ch).start()

        for b in range(NB):
            store(b, 0).wait()

    x_flat = x.reshape(B * S)
    vocab_rows = jnp.concatenate(
        [jnp.pad(vocab_table[i * (V // 4):(i + 1) * (V // 4)],
                 ((0, 0), (0, 128 - E))) for i in range(4)], axis=0)
    pos_flat = pos_table.reshape(2048 * 64)
    out_pad = k(x_flat, vocab_rows, pos_flat)
    return out_pad.reshape(B, S, 128)[:, :, :E]


# speedup vs baseline: 2.0894x; 2.0894x over previous
import functools

import jax
import jax.numpy as jnp
from jax import lax
from jax.experimental import pallas as pl
from jax.experimental.pallas import tpu as pltpu
from jax.experimental.pallas import tpu_sc as plsc

B, S, E = 1024, 200, 64
V = 1000000
NW = 32
TOK_PER_W = (B * S) // NW
CHUNK = 200
NCHUNK = TOK_PER_W // CHUNK
NB = 4


def kernel(x, vocab_table, pos_table):
    mesh = plsc.VectorSubcoreMesh(core_axis_name="c", subcore_axis_name="s")

    @functools.partial(
        pl.kernel,
        out_type=jax.ShapeDtypeStruct((B * S, 128), jnp.float32),
        scratch_types=[
            pltpu.VMEM((TOK_PER_W,), jnp.int32),
            pltpu.VMEM((S * E,), jnp.float32),
            pltpu.VMEM((NB, CHUNK, 128), jnp.float32),
            pltpu.SemaphoreType.DMA((NB,)),
            pltpu.SemaphoreType.DMA((NB,)),
        ],
        mesh=mesh,
        compiler_params=pltpu.CompilerParams(use_tc_tiling_on_sc=False),
    )
    def k(x_hbm, vocab_hbm, pos_hbm, out_hbm, idx_v, pos_v, buf, gsems,
          ssems):
        wid = lax.axis_index("s") * 2 + lax.axis_index("c")
        tok0 = wid * TOK_PER_W
        pltpu.sync_copy(x_hbm.at[pl.ds(tok0, TOK_PER_W)], idx_v)
        pltpu.sync_copy(pos_hbm.at[pl.ds(0, S * E)], pos_v)

        def gather(ch, b):
            return pltpu.make_async_copy(
                vocab_hbm.at[idx_v.at[pl.ds(ch * CHUNK, CHUNK)]],
                buf.at[b], gsems.at[b],
            )

        def store(b, ch):
            return pltpu.make_async_copy(
                buf.at[b], out_hbm.at[pl.ds(tok0 + ch * CHUNK, CHUNK)],
                ssems.at[b],
            )

        for b in range(NB):
            gather(b, b).start()

        for ch in range(NCHUNK):
            b = ch % NB
            if 0 < ch and ch - 1 + NB < NCHUNK:
                pb = (ch - 1) % NB
                store(pb, ch - 1).wait()
                gather(ch - 1 + NB, pb).start()
            gather(ch, b).wait()

            @plsc.parallel_loop(0, CHUNK, 1, unroll=4)
            def add_body(i, b=b):
                for v in range(E // 16):
                    buf[b, i, pl.ds(v * 16, 16)] = (
                        buf[b, i, pl.ds(v * 16, 16)]
                        + pos_v[pl.ds(i * E + v * 16, 16)]
                    )
            store(b, ch).start()

        for b in range(NB):
            store(b, 0).wait()

    x_flat = x.reshape(B * S)
    vocab_rows = jnp.pad(vocab_table, ((0, 0), (0, 128 - E)))
    pos_flat = pos_table.reshape(2048 * 64)
    out_pad = k(x_flat, vocab_rows, pos_flat)
    return out_pad.reshape(B, S, 128)[:, :, :E]
